# baseline (speedup 1.0000x reference)
import functools

import jax
import jax.numpy as jnp
from jax import lax
from jax.experimental import pallas as pl
from jax.experimental.pallas import tpu as pltpu
from jax.experimental.pallas import tpu_sc as plsc

_NUM_SC = 2
_NUM_SUBCORES = 16


def _sc_pe_broadcast(col_table, row_table, B, H, W, C):
    half = col_table.shape[1]
    HW = H * W
    N_RCHUNK = 8
    N_BGROUP = 4
    r_chunk = HW // N_RCHUNK
    h_per_chunk = r_chunk // W
    b_group = B // N_BGROUP

    mesh = plsc.VectorSubcoreMesh(
        core_axis_name="c", subcore_axis_name="s",
        num_cores=_NUM_SC, num_subcores=_NUM_SUBCORES)

    @functools.partial(
        pl.kernel,
        out_type=jax.ShapeDtypeStruct((B, HW, C), jnp.float32),
        mesh=mesh,
        scratch_types=[
            pltpu.VMEM((r_chunk, C), jnp.float32),
            pltpu.SemaphoreType.DMA,
        ],
    )
    def fanout(col_hbm, row_hbm, out_hbm, slice_v, sem):
        wid = lax.axis_index("s") * _NUM_SC + lax.axis_index("c")
        rchunk_id = lax.rem(wid, N_RCHUNK)
        bgroup_id = lax.div(wid, N_RCHUNK)
        base_r = rchunk_id * r_chunk
        base_b = bgroup_id * b_group
        h0 = rchunk_id * h_per_chunk

        for h5 in range(h_per_chunk):
            pltpu.sync_copy(
                col_hbm, slice_v.at[pl.ds(h5 * W, W), pl.ds(0, half)])

        def _fire_row(r, _):
            pltpu.make_async_copy(
                row_hbm.at[pl.ds(h0 + lax.div(r, W), 1)],
                slice_v.at[pl.ds(r, 1), pl.ds(half, half)],
                sem).start()
            return _

        def _drain_row(r, _):
            pltpu.make_async_copy(
                row_hbm.at[pl.ds(0, 1)],
                slice_v.at[pl.ds(0, 1), pl.ds(half, half)],
                sem).wait()
            return _

        lax.fori_loop(0, r_chunk, _fire_row, None)
        lax.fori_loop(0, r_chunk, _drain_row, None)

        copies = [
            pltpu.make_async_copy(
                slice_v, out_hbm.at[base_b + k].at[pl.ds(base_r, r_chunk)],
                sem)
            for k in range(b_group)
        ]
        for cp in copies:
            cp.start()
        for cp in copies:
            cp.wait()

    return fanout(col_table, row_table)


def kernel(image_feature, col_table, row_table):
    B, C, H, W = image_feature.shape
    out = _sc_pe_broadcast(col_table, row_table, B, H, W, C)
    return out.reshape(B, H, W, C).transpose(0, 3, 1, 2)


# speedup vs baseline: 1.9022x; 1.9022x over previous
import functools

import jax
import jax.numpy as jnp
from jax import lax
from jax.experimental import pallas as pl
from jax.experimental.pallas import tpu as pltpu
from jax.experimental.pallas import tpu_sc as plsc

_NUM_SC = 2
_NUM_SUBCORES = 16


def _pe_build_kernel(col_ref, row_ref, pe_ref):
    W, half = col_ref.shape
    H = row_ref.shape[0]
    col = col_ref[...]
    row = row_ref[...]
    pe_ref[:, :, :half] = jnp.broadcast_to(col[None, :, :], (H, W, half))
    pe_ref[:, :, half:] = jnp.broadcast_to(row[:, None, :], (H, W, half))


def _build_pe(col_table, row_table, H, W, C):
    return pl.pallas_call(
        _pe_build_kernel,
        out_shape=jax.ShapeDtypeStruct((H, W, C), jnp.float32),
    )(col_table, row_table)


def _sc_fanout(pe, B):
    HW, C = pe.shape
    N_RCHUNK = 8
    N_BGROUP = 4
    r_chunk = HW // N_RCHUNK
    b_group = B // N_BGROUP

    mesh = plsc.VectorSubcoreMesh(
        core_axis_name="c", subcore_axis_name="s",
        num_cores=_NUM_SC, num_subcores=_NUM_SUBCORES)

    @functools.partial(
        pl.kernel,
        out_type=jax.ShapeDtypeStruct((B, HW, C), jnp.float32),
        mesh=mesh,
        scratch_types=[
            pltpu.VMEM((r_chunk, C), jnp.float32),
            pltpu.SemaphoreType.DMA,
        ],
    )
    def fanout(pe_hbm, out_hbm, slice_v, sem):
        wid = lax.axis_index("s") * _NUM_SC + lax.axis_index("c")
        rchunk_id = lax.rem(wid, N_RCHUNK)
        bgroup_id = lax.div(wid, N_RCHUNK)
        base_r = rchunk_id * r_chunk
        base_b = bgroup_id * b_group
        pltpu.sync_copy(pe_hbm.at[pl.ds(base_r, r_chunk)], slice_v)

        def _fire(k, carry):
            pltpu.make_async_copy(
                slice_v, out_hbm.at[base_b + k].at[pl.ds(base_r, r_chunk)],
                sem).start()
            return carry

        def _drain(k, carry):
            pltpu.make_async_copy(
                slice_v, out_hbm.at[base_b].at[pl.ds(base_r, r_chunk)],
                sem).wait()
            return carry

        lax.fori_loop(0, b_group, _fire, None)
        lax.fori_loop(0, b_group, _drain, None)

    return fanout(pe)


def kernel(image_feature, col_table, row_table):
    B, C, H, W = image_feature.shape
    pe = _build_pe(col_table, row_table, H, W, C)
    out = _sc_fanout(pe.reshape(H * W, C), B)
    return out.reshape(B, H, W, C).transpose(0, 3, 1, 2)


# speedup vs baseline: 1.9259x; 1.0124x over previous
import functools

import jax
import jax.numpy as jnp
from jax import lax
from jax.experimental import pallas as pl
from jax.experimental.pallas import tpu as pltpu
from jax.experimental.pallas import tpu_sc as plsc

_NUM_SC = 2
_NUM_SUBCORES = 16


def _pe_build_kernel(col_ref, row_ref, pe_ref):
    W, half = col_ref.shape
    H = row_ref.shape[0]
    col = col_ref[...]
    row = row_ref[...]
    pe_ref[:, :, :half] = jnp.broadcast_to(col[None, :, :], (H, W, half))
    pe_ref[:, :, half:] = jnp.broadcast_to(row[:, None, :], (H, W, half))


def _build_pe(col_table, row_table, H, W, C):
    return pl.pallas_call(
        _pe_build_kernel,
        out_shape=jax.ShapeDtypeStruct((H, W, C), jnp.float32),
    )(col_table, row_table)


def _sc_fanout(pe, B):
    HW, C = pe.shape
    N_RCHUNK = 8
    N_BGROUP = 4
    r_chunk = HW // N_RCHUNK
    b_group = B // N_BGROUP

    mesh = plsc.VectorSubcoreMesh(
        core_axis_name="c", subcore_axis_name="s",
        num_cores=_NUM_SC, num_subcores=_NUM_SUBCORES)

    @functools.partial(
        pl.kernel,
        out_type=jax.ShapeDtypeStruct((B, HW, C), jnp.float32),
        mesh=mesh,
        scratch_types=[
            pltpu.VMEM((r_chunk, C), jnp.float32),
            pltpu.SemaphoreType.DMA,
        ],
        compiler_params=pltpu.CompilerParams(skip_device_barrier=True),
    )
    def fanout(pe_hbm, out_hbm, slice_v, sem):
        wid = lax.axis_index("s") * _NUM_SC + lax.axis_index("c")
        rchunk_id = lax.rem(wid, N_RCHUNK)
        bgroup_id = lax.div(wid, N_RCHUNK)
        base_r = rchunk_id * r_chunk
        base_b = bgroup_id * b_group
        pltpu.sync_copy(pe_hbm.at[pl.ds(base_r, r_chunk)], slice_v)

        copies = [
            pltpu.make_async_copy(
                slice_v, out_hbm.at[base_b + k].at[pl.ds(base_r, r_chunk)],
                sem)
            for k in range(b_group)
        ]
        for cp in copies:
            cp.start()
        for cp in copies:
            cp.wait()

    return fanout(pe)


def kernel(image_feature, col_table, row_table):
    B, C, H, W = image_feature.shape
    pe = _build_pe(col_table, row_table, H, W, C)
    out = _sc_fanout(pe.reshape(H * W, C), B)
    return out.reshape(B, H, W, C).transpose(0, 3, 1, 2)


# speedup vs baseline: 1.9260x; 1.0001x over previous
import functools

import jax
import jax.numpy as jnp
from jax import lax
from jax.experimental import pallas as pl
from jax.experimental.pallas import tpu as pltpu
from jax.experimental.pallas import tpu_sc as plsc

_NUM_SC = 2
_NUM_SUBCORES = 16


def _pe_build_kernel(col_ref, row_ref, pe_ref):
    W, half = col_ref.shape
    H = row_ref.shape[0]
    col = col_ref[...]
    row = row_ref[...]
    pe_ref[:, :, :half] = jnp.broadcast_to(col[None, :, :], (H, W, half))
    pe_ref[:, :, half:] = jnp.broadcast_to(row[:, None, :], (H, W, half))


def _build_pe(col_table, row_table, H, W, C):
    return pl.pallas_call(
        _pe_build_kernel,
        out_shape=jax.ShapeDtypeStruct((H, W, C), jnp.float32),
    )(col_table, row_table)


def _sc_fanout(pe, B):
    HW, C = pe.shape
    N_RCHUNK = 8
    N_BGROUP = 4
    r_chunk = HW // N_RCHUNK
    b_group = B // N_BGROUP

    mesh = plsc.VectorSubcoreMesh(
        core_axis_name="c", subcore_axis_name="s",
        num_cores=_NUM_SC, num_subcores=_NUM_SUBCORES)

    @functools.partial(
        pl.kernel,
        out_type=jax.ShapeDtypeStruct((B, HW, C), jnp.float32),
        mesh=mesh,
        scratch_types=[
            pltpu.VMEM((r_chunk, C), jnp.float32),
            pltpu.SemaphoreType.DMA,
        ],
    )
    def fanout(pe_hbm, out_hbm, slice_v, sem):
        wid = lax.axis_index("s") * _NUM_SC + lax.axis_index("c")
        rchunk_id = lax.rem(wid, N_RCHUNK)
        bgroup_id = lax.div(wid, N_RCHUNK)
        base_r = rchunk_id * r_chunk
        base_b = bgroup_id * b_group
        pltpu.sync_copy(pe_hbm.at[pl.ds(base_r, r_chunk)], slice_v)

        copies = [
            pltpu.make_async_copy(
                slice_v, out_hbm.at[base_b + k].at[pl.ds(base_r, r_chunk)],
                sem)
            for k in range(b_group)
        ]
        for cp in copies:
            cp.start()
        for cp in copies:
            cp.wait()

    return fanout(pe)


def kernel(image_feature, col_table, row_table):
    B, C, H, W = image_feature.shape
    pe = _build_pe(col_table, row_table, H, W, C)
    out = _sc_fanout(pe.reshape(H * W, C), B)
    return out.reshape(B, H, W, C).transpose(0, 3, 1, 2)
